# baseline (speedup 1.0000x reference)
import functools

import jax
import jax.numpy as jnp
from jax import lax
from jax.experimental import pallas as pl
from jax.experimental.pallas import tpu as pltpu
from jax.experimental.pallas import tpu_sc as plsc

N_NODES = 10000
EMB = 128
N_EDGES = 320000

NC = 2
NS = 16
NW = NC * NS
LANES = 16

CHUNK = 80
CHUNKS_TOTAL = N_EDGES // CHUNK
CHUNKS_PER_W = CHUNKS_TOTAL // NW
STAGES = 5
SCHUNKS = CHUNKS_PER_W // STAGES
HALF = CHUNK // 2
ROWS_PER_TILE = 624
ROWS_REMAINDER = N_NODES - NS * ROWS_PER_TILE


def _sc_partial_body(row_hbm, col_hbm, val_hbm, emb_hbm, zeros_hbm, out_hbm,
                     rowv, colv, valv, bufa, bufb, bufc, acc,
                     gsa, gsb, gsc, ssa, ssb, ssc):
    cid = lax.axis_index("c")
    sid = lax.axis_index("s")
    wid = cid * NS + sid

    zslice = pl.ds(sid * ROWS_PER_TILE, ROWS_PER_TILE)
    rslice = pl.ds(NS * ROWS_PER_TILE, ROWS_REMAINDER)
    pltpu.sync_copy(zeros_hbm.at[zslice], acc.at[zslice])

    @pl.when(sid == NS - 1)
    def _zero_tail():
        pltpu.sync_copy(zeros_hbm.at[rslice], acc.at[rslice])

    plsc.subcore_barrier()

    def scale(b, i, vref):
        def scale_block(rb, c2):
            vvals = vref[i, pl.ds(rb * LANES, LANES)]
            for k in range(LANES):
                s = vvals[k]
                r = rb * LANES + k
                for j in range(EMB // LANES):
                    sl = pl.ds(j * LANES, LANES)
                    b[r, sl] = b[r, sl] * s
            return c2
        lax.fori_loop(0, CHUNK // LANES, scale_block, 0)

    def gather(i, b, gs):
        pltpu.async_copy(emb_hbm.at[colv.at[i, 0]], b.at[pl.ds(0, HALF)], gs)
        pltpu.async_copy(emb_hbm.at[colv.at[i, 1]], b.at[pl.ds(HALF, HALF)], gs)

    def scatter(i, b, ss):
        pltpu.async_copy(b.at[pl.ds(0, HALF)], acc.at[rowv.at[i, 0]], ss,
                         add=True)
        pltpu.async_copy(b.at[pl.ds(HALF, HALF)], acc.at[rowv.at[i, 1]], ss,
                         add=True)

    def wait(b, s):
        pltpu.make_async_copy(emb_hbm.at[pl.ds(0, CHUNK)], b, s).wait()

    def stage_body(p, carry0):
        pltpu.sync_copy(row_hbm.at[wid, p], rowv)
        pltpu.sync_copy(col_hbm.at[wid, p], colv)
        pltpu.sync_copy(val_hbm.at[wid, p], valv)

        bufs = (bufa, bufb, bufc)
        gsems = (gsa, gsb, gsc)
        ssems = (ssa, ssb, ssc)

        def process(i, t, trailing):
            b, gs, ss = bufs[t], gsems[t], ssems[t]
            wait(b, gs)
            scale(b, i, valv)
            scatter(i, b, ss)
            if trailing:
                x = (t + 2) % 3
                wait(bufs[x], ssems[x])
                gather(i + 2, bufs[x], gsems[x])

        gather(0, bufa, gsa)
        gather(1, bufb, gsb)
        gather(2, bufc, gsc)
        process(0, 0, False)

        def chunk_trip(k, carry):
            i0 = 3 * k + 1
            process(i0, 1, True)
            process(i0 + 1, 2, True)
            process(i0 + 2, 0, True)
            return carry

        lax.fori_loop(0, (SCHUNKS - 4) // 3, chunk_trip, 0)

        process(SCHUNKS - 3, 1, True)
        process(SCHUNKS - 2, 2, False)
        process(SCHUNKS - 1, 0, False)
        wait(bufa, ssa)
        wait(bufb, ssb)
        wait(bufc, ssc)
        return carry0

    lax.fori_loop(0, STAGES, stage_body, 0)

    plsc.subcore_barrier()

    pltpu.sync_copy(acc.at[zslice], out_hbm.at[cid, zslice])

    @pl.when(sid == NS - 1)
    def _dump_tail():
        pltpu.sync_copy(acc.at[rslice], out_hbm.at[cid, rslice])


_sc_partial = functools.partial(
    pl.kernel,
    out_type=jax.ShapeDtypeStruct((NC, N_NODES, EMB), jnp.float32),
    mesh=plsc.VectorSubcoreMesh(core_axis_name="c", subcore_axis_name="s"),
    scratch_types=[
        pltpu.VMEM((SCHUNKS, 2, HALF), jnp.int32),
        pltpu.VMEM((SCHUNKS, 2, HALF), jnp.int32),
        pltpu.VMEM((SCHUNKS, CHUNK), jnp.float32),
        pltpu.VMEM((CHUNK, EMB), jnp.float32),
        pltpu.VMEM((CHUNK, EMB), jnp.float32),
        pltpu.VMEM((CHUNK, EMB), jnp.float32),
        pltpu.VMEM_SHARED((N_NODES, EMB), jnp.float32),
        pltpu.SemaphoreType.DMA,
        pltpu.SemaphoreType.DMA,
        pltpu.SemaphoreType.DMA,
        pltpu.SemaphoreType.DMA,
        pltpu.SemaphoreType.DMA,
        pltpu.SemaphoreType.DMA,
    ],
)(_sc_partial_body)


def _combine_body(p_ref, emb_ref, w_ref, out_ref):
    x = 0.8 * (p_ref[0] + p_ref[1]) + 0.2 * emb_ref[...]
    y = jnp.dot(x, w_ref[...], preferred_element_type=jnp.float32)
    out_ref[...] = jax.nn.sigmoid(y)


_BR = 1000


def _combine(partials, emb, w):
    grid = N_NODES // _BR
    return pl.pallas_call(
        _combine_body,
        grid=(grid,),
        in_specs=[
            pl.BlockSpec((NC, _BR, EMB), lambda i: (0, i, 0)),
            pl.BlockSpec((_BR, EMB), lambda i: (i, 0)),
            pl.BlockSpec((EMB, EMB), lambda i: (0, 0)),
        ],
        out_specs=pl.BlockSpec((_BR, EMB), lambda i: (i, 0)),
        out_shape=jax.ShapeDtypeStruct((N_NODES, EMB), jnp.float32),
    )(partials, emb, w)


def kernel(edge_index, edge_values, user_embedding, item_embedding, W0, W1, W2):
    row2d = edge_index[0].reshape(NW, STAGES, SCHUNKS, 2, HALF)
    col2d = edge_index[1].reshape(NW, STAGES, SCHUNKS, 2, HALF)
    val2d = edge_values.reshape(NW, STAGES, SCHUNKS, CHUNK)
    zeros = jnp.zeros((N_NODES, EMB), jnp.float32)

    emb = jnp.concatenate([user_embedding, item_embedding], axis=0)
    outs = [emb]
    for w in (W0, W1, W2):
        partials = _sc_partial(row2d, col2d, val2d, emb, zeros)
        emb = _combine(partials, emb, w)
        outs.append(emb)

    all_e = jnp.concatenate(outs, axis=1)
    n_users = user_embedding.shape[0]
    return (all_e[:n_users], all_e[n_users:])


# speedup vs baseline: 1.0835x; 1.0835x over previous
import functools

import jax
import jax.numpy as jnp
from jax import lax
from jax.experimental import pallas as pl
from jax.experimental.pallas import tpu as pltpu
from jax.experimental.pallas import tpu_sc as plsc

N_NODES = 10000
EMB = 128
N_EDGES = 320000

NC = 2
NS = 16
NW = NC * NS
LANES = 16

CHUNK = 80
CHUNKS_TOTAL = N_EDGES // CHUNK
CHUNKS_PER_W = CHUNKS_TOTAL // NW
STAGES = 5
SCHUNKS = CHUNKS_PER_W // STAGES
ROWS_PER_TILE = 624
ROWS_REMAINDER = N_NODES - NS * ROWS_PER_TILE


def _sc_partial_body(row_hbm, col_hbm, val_hbm, emb_hbm, zeros_hbm, out_hbm,
                     rowv, colv, valv, bufa, bufb, bufc, acc,
                     gsa, gsb, gsc, ssa, ssb, ssc, zsem, isem):
    cid = lax.axis_index("c")
    sid = lax.axis_index("s")
    wid = cid * NS + sid

    zslice = pl.ds(sid * ROWS_PER_TILE, ROWS_PER_TILE)
    rslice = pl.ds(NS * ROWS_PER_TILE, ROWS_REMAINDER)
    pltpu.async_copy(zeros_hbm.at[zslice], acc.at[zslice], zsem)

    @pl.when(sid == NS - 1)
    def _zero_tail():
        pltpu.async_copy(zeros_hbm.at[rslice], acc.at[rslice], zsem)

    def fetch(p):
        pltpu.async_copy(row_hbm.at[wid, p], rowv, isem)
        pltpu.async_copy(col_hbm.at[wid, p], colv, isem)
        pltpu.async_copy(val_hbm.at[wid, p], valv, isem)

    def fetch_wait():
        pltpu.make_async_copy(row_hbm.at[0, 0], rowv, isem).wait()
        pltpu.make_async_copy(col_hbm.at[0, 0], colv, isem).wait()
        pltpu.make_async_copy(val_hbm.at[0, 0], valv, isem).wait()

    fetch(0)

    pltpu.make_async_copy(zeros_hbm.at[zslice], acc.at[zslice], zsem).wait()

    @pl.when(sid == NS - 1)
    def _zero_tail_wait():
        pltpu.make_async_copy(zeros_hbm.at[rslice], acc.at[rslice],
                              zsem).wait()

    plsc.subcore_barrier()

    def scale(i, b):
        def scale_block(rb, c2):
            vvals = valv[i, pl.ds(rb * LANES, LANES)]
            for k in range(LANES):
                s = vvals[k]
                r = rb * LANES + k
                for j in range(EMB // LANES):
                    sl = pl.ds(j * LANES, LANES)
                    b[r, sl] = b[r, sl] * s
            return c2
        lax.fori_loop(0, CHUNK // LANES, scale_block, 0)

    def gather(i, b, gs):
        pltpu.async_copy(emb_hbm.at[colv.at[i]], b, gs)

    def scatter(i, b, ss):
        pltpu.async_copy(b, acc.at[rowv.at[i]], ss, add=True)

    def wait(b, s):
        pltpu.make_async_copy(emb_hbm.at[pl.ds(0, CHUNK)], b, s).wait()

    def stage_body(p, carry0):
        @pl.when(p > 0)
        def _fetch_stage():
            fetch(p)
        fetch_wait()

        bufs = (bufa, bufb, bufc)
        gsems = (gsa, gsb, gsc)
        ssems = (ssa, ssb, ssc)

        def process(i, t, trailing):
            b, gs, ss = bufs[t], gsems[t], ssems[t]
            wait(b, gs)
            scale(i, b)
            scatter(i, b, ss)
            if trailing:
                x = (t + 2) % 3
                wait(bufs[x], ssems[x])
                gather(i + 2, bufs[x], gsems[x])

        gather(0, bufa, gsa)
        gather(1, bufb, gsb)
        gather(2, bufc, gsc)
        process(0, 0, False)

        def chunk_trip(k, carry):
            i0 = 3 * k + 1
            process(i0, 1, True)
            process(i0 + 1, 2, True)
            process(i0 + 2, 0, True)
            return carry

        lax.fori_loop(0, (SCHUNKS - 4) // 3, chunk_trip, 0)

        process(SCHUNKS - 3, 1, True)
        process(SCHUNKS - 2, 2, False)
        process(SCHUNKS - 1, 0, False)
        wait(bufa, ssa)
        wait(bufb, ssb)
        wait(bufc, ssc)
        return carry0

    lax.fori_loop(0, STAGES, stage_body, 0)

    plsc.subcore_barrier()

    pltpu.sync_copy(acc.at[zslice], out_hbm.at[cid, zslice])

    @pl.when(sid == NS - 1)
    def _dump_tail():
        pltpu.sync_copy(acc.at[rslice], out_hbm.at[cid, rslice])


_sc_partial = functools.partial(
    pl.kernel,
    out_type=jax.ShapeDtypeStruct((NC, N_NODES, EMB), jnp.float32),
    mesh=plsc.VectorSubcoreMesh(core_axis_name="c", subcore_axis_name="s"),
    scratch_types=[
        pltpu.VMEM((SCHUNKS, CHUNK), jnp.int32),
        pltpu.VMEM((SCHUNKS, CHUNK), jnp.int32),
        pltpu.VMEM((SCHUNKS, CHUNK), jnp.float32),
        pltpu.VMEM((CHUNK, EMB), jnp.float32),
        pltpu.VMEM((CHUNK, EMB), jnp.float32),
        pltpu.VMEM((CHUNK, EMB), jnp.float32),
        pltpu.VMEM_SHARED((N_NODES, EMB), jnp.float32),
        pltpu.SemaphoreType.DMA,
        pltpu.SemaphoreType.DMA,
        pltpu.SemaphoreType.DMA,
        pltpu.SemaphoreType.DMA,
        pltpu.SemaphoreType.DMA,
        pltpu.SemaphoreType.DMA,
        pltpu.SemaphoreType.DMA,
        pltpu.SemaphoreType.DMA,
    ],
)(_sc_partial_body)


def _combine_body(p_ref, emb_ref, w_ref, out_ref):
    x = 0.8 * (p_ref[0] + p_ref[1]) + 0.2 * emb_ref[...]
    y = jnp.dot(x, w_ref[...], preferred_element_type=jnp.float32)
    out_ref[...] = jax.nn.sigmoid(y)


_BR = 1000


def _combine(partials, emb, w):
    grid = N_NODES // _BR
    return pl.pallas_call(
        _combine_body,
        grid=(grid,),
        in_specs=[
            pl.BlockSpec((NC, _BR, EMB), lambda i: (0, i, 0)),
            pl.BlockSpec((_BR, EMB), lambda i: (i, 0)),
            pl.BlockSpec((EMB, EMB), lambda i: (0, 0)),
        ],
        out_specs=pl.BlockSpec((_BR, EMB), lambda i: (i, 0)),
        out_shape=jax.ShapeDtypeStruct((N_NODES, EMB), jnp.float32),
    )(partials, emb, w)


def kernel(edge_index, edge_values, user_embedding, item_embedding, W0, W1, W2):
    row2d = edge_index[0].reshape(NW, STAGES, SCHUNKS, CHUNK)
    col2d = edge_index[1].reshape(NW, STAGES, SCHUNKS, CHUNK)
    val2d = edge_values.reshape(NW, STAGES, SCHUNKS, CHUNK)
    zeros = jnp.zeros((N_NODES, EMB), jnp.float32)

    emb = jnp.concatenate([user_embedding, item_embedding], axis=0)
    outs = [emb]
    for w in (W0, W1, W2):
        partials = _sc_partial(row2d, col2d, val2d, emb, zeros)
        emb = _combine(partials, emb, w)
        outs.append(emb)

    all_e = jnp.concatenate(outs, axis=1)
    n_users = user_embedding.shape[0]
    return (all_e[:n_users], all_e[n_users:])
